# baseline (speedup 1.0000x reference)
import jax
import jax.numpy as jnp
from jax.experimental import pallas as pl
from jax.experimental.pallas import tpu as pltpu

NBITS = 256
HIDDEN = 256
SPATIAL = 32
B = 128
C = 128
HW = SPATIAL * SPATIAL


def _body(msg_ref, we_ref, wo_ref, lat_ref, out_ref):
    lat = lat_ref[0]
    we = we_ref[...]
    d = wo_ref[...] - we
    base = jnp.sum(we, axis=0, keepdims=True)
    aux = jax.lax.dot_general(
        msg_ref[0], d, (((1,), (0,)), ((), ())),
        preferred_element_type=jnp.float32) + base
    out_ref[0, :C, :] = lat
    out_ref[0, C:, :] = jnp.broadcast_to(aux.reshape(HIDDEN, 1), (HIDDEN, HW))


def kernel(latents, msg, W):
    lat3 = latents.reshape(B, C, HW)
    msg_f = msg.astype(jnp.float32).reshape(B, 1, NBITS)
    we = W[0::2]
    wo = W[1::2]
    out = pl.pallas_call(
        _body,
        grid=(B,),
        in_specs=[
            pl.BlockSpec((1, 1, NBITS), lambda b: (b, 0, 0)),
            pl.BlockSpec((NBITS, HIDDEN), lambda b: (0, 0)),
            pl.BlockSpec((NBITS, HIDDEN), lambda b: (0, 0)),
            pl.BlockSpec((1, C, HW), lambda b: (b, 0, 0)),
        ],
        out_specs=pl.BlockSpec((1, C + HIDDEN, HW), lambda b: (b, 0, 0)),
        out_shape=jax.ShapeDtypeStruct((B, C + HIDDEN, HW), jnp.float32),
    )(msg_f, we, wo, lat3)
    return out.reshape(B, C + HIDDEN, SPATIAL, SPATIAL)
